# baseline (speedup 1.0000x reference)
import jax
import jax.numpy as jnp
from jax.experimental import pallas as pl
from jax.experimental.pallas import tpu as pltpu
from jax.experimental.pallas import tpu_sc as plsc

ROWS = 8
FPAD = 32


def kernel(x, weight):
    batch, fields = x.shape
    dim = weight.shape[1]
    xi = x.astype(jnp.int32)
    idx = jnp.pad(
        jnp.concatenate([xi, xi[:, : FPAD - fields]], axis=1),
        ((0, 0), (0, 128 - FPAD)),
    )
    w128 = jnp.pad(weight, ((0, 0), (0, 128 - dim)))

    mesh = plsc.VectorSubcoreMesh(core_axis_name="core", subcore_axis_name="subcore")

    @pl.kernel(
        out_type=jax.ShapeDtypeStruct((batch * FPAD, 128), weight.dtype),
        mesh=mesh,
        scratch_types=[pltpu.SemaphoreType.DMA],
        compiler_params=pltpu.CompilerParams(use_tc_tiling_on_sc=False),
    )
    def gather_kernel(w_hbm, i_hbm, o_hbm, sem):
        def body(i_vmem, o_vmem):
            @pl.loop(0, ROWS)
            def _(r):
                pltpu.async_copy(
                    w_hbm.at[i_vmem.at[r, pl.ds(0, FPAD)]],
                    o_vmem.at[pl.ds(r * FPAD, FPAD)],
                    sem,
                )

            pltpu.make_async_copy(o_hbm.at[pl.ds(0, ROWS * FPAD)], o_vmem, sem).wait()

        pltpu.emit_pipeline(
            body,
            grid=(batch // ROWS,),
            in_specs=[pl.BlockSpec((ROWS, 128), index_map=lambda i: (i, 0))],
            out_specs=[pl.BlockSpec((ROWS * FPAD, 128), index_map=lambda i: (i, 0))],
            core_axis_name=("core", "subcore"),
            dimension_semantics=(pltpu.PARALLEL,),
        )(i_hbm, o_hbm)

    out = gather_kernel(w128, idx)
    return out.reshape(batch, FPAD, 128)[:, :fields, :dim]


# speedup vs baseline: 1.0444x; 1.0444x over previous
import jax
import jax.numpy as jnp
from jax import lax
from jax.experimental import pallas as pl
from jax.experimental.pallas import tpu as pltpu
from jax.experimental.pallas import tpu_sc as plsc

NUM_CORES = 2
NUM_SUBCORES = 16
NUM_WORKERS = NUM_CORES * NUM_SUBCORES

RSTEP = 16
FPAD = 32
SLAB = RSTEP * FPAD


def kernel(x, weight):
    batch, fields = x.shape
    dim = weight.shape[1]
    xi = x.astype(jnp.int32)
    idx = jnp.pad(
        jnp.concatenate([xi, xi[:, : FPAD - fields]], axis=1),
        ((0, 0), (0, 128 - FPAD)),
    )

    rows_pw = batch // NUM_WORKERS
    steps = rows_pw // RSTEP

    mesh = plsc.VectorSubcoreMesh(core_axis_name="core", subcore_axis_name="subcore")

    @pl.kernel(
        out_type=jax.ShapeDtypeStruct((batch * FPAD, 128), weight.dtype),
        mesh=mesh,
        scratch_types=[
            pltpu.VMEM((rows_pw, FPAD), jnp.int32),
            pltpu.VMEM((SLAB, dim), jnp.float32),
            pltpu.VMEM((SLAB, dim), jnp.float32),
            pltpu.SemaphoreType.DMA,
            pltpu.SemaphoreType.DMA,
            pltpu.SemaphoreType.DMA,
            pltpu.SemaphoreType.DMA,
        ],
        compiler_params=pltpu.CompilerParams(use_tc_tiling_on_sc=False),
    )
    def gather_kernel(w_hbm, i_hbm, o_hbm, idx_v, rows0, rows1, sg0, sg1, so0, so1):
        wid = lax.axis_index("subcore") * NUM_CORES + lax.axis_index("core")
        row0 = wid * rows_pw
        pltpu.sync_copy(i_hbm.at[pl.ds(row0, rows_pw), pl.ds(0, FPAD)], idx_v)

        def fire(step, rows_v, sg):
            @pl.loop(0, RSTEP)
            def _(r):
                pltpu.async_copy(
                    w_hbm.at[idx_v.at[step * RSTEP + r, pl.ds(0, FPAD)]],
                    rows_v.at[pl.ds(r * FPAD, FPAD)],
                    sg,
                )

        def drain_gather(rows_v, sg):
            pltpu.make_async_copy(w_hbm.at[pl.ds(0, SLAB)], rows_v, sg).wait()

        def store(step, rows_v, so):
            pltpu.async_copy(
                rows_v,
                o_hbm.at[pl.ds((row0 + step * RSTEP) * FPAD, SLAB), pl.ds(0, dim)],
                so,
            )

        def wait_store(rows_v, so):
            pltpu.make_async_copy(
                rows_v, o_hbm.at[pl.ds(0, SLAB), pl.ds(0, dim)], so
            ).wait()

        @pl.loop(0, steps, step=2)
        def _(s):
            for b, rows_v, sg, so in ((0, rows0, sg0, so0), (1, rows1, sg1, so1)):
                ss = s + b

                @pl.when(ss >= 2)
                def _():
                    wait_store(rows_v, so)

                fire(ss, rows_v, sg)
                drain_gather(rows_v, sg)
                store(ss, rows_v, so)

        wait_store(rows0, so0)
        wait_store(rows1, so1)

    out = gather_kernel(weight, idx)
    return out.reshape(batch, FPAD, 128)[:, :fields, :dim]


# speedup vs baseline: 1.0492x; 1.0047x over previous
import jax
import jax.numpy as jnp
from jax import lax
from jax.experimental import pallas as pl
from jax.experimental.pallas import tpu as pltpu
from jax.experimental.pallas import tpu_sc as plsc

NUM_CORES = 2
NUM_SUBCORES = 16
NUM_WORKERS = NUM_CORES * NUM_SUBCORES

RSTEP = 8
NBUF = 4
FPAD = 32
SLAB = RSTEP * FPAD


def kernel(x, weight):
    batch, fields = x.shape
    dim = weight.shape[1]
    xi = x.astype(jnp.int32)
    idx = jnp.pad(
        jnp.concatenate([xi, xi[:, : FPAD - fields]], axis=1),
        ((0, 0), (0, 128 - FPAD)),
    )

    rows_pw = batch // NUM_WORKERS
    steps = rows_pw // RSTEP

    mesh = plsc.VectorSubcoreMesh(core_axis_name="core", subcore_axis_name="subcore")

    @pl.kernel(
        out_type=jax.ShapeDtypeStruct((batch * FPAD, 128), weight.dtype),
        mesh=mesh,
        scratch_types=[
            pltpu.VMEM((rows_pw, FPAD), jnp.int32),
        ]
        + [pltpu.VMEM((SLAB, dim), jnp.float32) for _ in range(NBUF)]
        + [pltpu.SemaphoreType.DMA for _ in range(2 * NBUF)],
        compiler_params=pltpu.CompilerParams(use_tc_tiling_on_sc=False),
    )
    def gather_kernel(w_hbm, i_hbm, o_hbm, idx_v, *bufs_and_sems):
        rows = bufs_and_sems[:NBUF]
        sgs = bufs_and_sems[NBUF : 2 * NBUF]
        sos = bufs_and_sems[2 * NBUF : 3 * NBUF]
        wid = lax.axis_index("subcore") * NUM_CORES + lax.axis_index("core")
        row0 = wid * rows_pw
        pltpu.sync_copy(i_hbm.at[pl.ds(row0, rows_pw), pl.ds(0, FPAD)], idx_v)

        def fire(step, rows_v, sg):
            @pl.loop(0, RSTEP)
            def _(r):
                pltpu.async_copy(
                    w_hbm.at[idx_v.at[step * RSTEP + r, pl.ds(0, FPAD)]],
                    rows_v.at[pl.ds(r * FPAD, FPAD)],
                    sg,
                )

        def drain_gather(rows_v, sg):
            pltpu.make_async_copy(w_hbm.at[pl.ds(0, SLAB)], rows_v, sg).wait()

        def store(step, rows_v, so):
            pltpu.async_copy(
                rows_v,
                o_hbm.at[pl.ds((row0 + step * RSTEP) * FPAD, SLAB), pl.ds(0, dim)],
                so,
            )

        def wait_store(rows_v, so):
            pltpu.make_async_copy(
                rows_v, o_hbm.at[pl.ds(0, SLAB), pl.ds(0, dim)], so
            ).wait()

        fire(0, rows[0], sgs[0])

        @pl.loop(0, steps, step=NBUF)
        def _(s):
            for j in range(NBUF):
                ss = s + j
                jn = (j + 1) % NBUF

                @pl.when(ss + 1 < steps)
                def _():
                    @pl.when(ss + 1 >= NBUF)
                    def _():
                        wait_store(rows[jn], sos[jn])

                    fire(ss + 1, rows[jn], sgs[jn])

                drain_gather(rows[j], sgs[j])
                store(ss, rows[j], sos[j])

        for j in range(NBUF):
            wait_store(rows[j], sos[j])

    out = gather_kernel(weight, idx)
    return out.reshape(batch, FPAD, 128)[:, :fields, :dim]
